# baseline (speedup 1.0000x reference)
import jax
import jax.numpy as jnp
from jax import lax
from jax.experimental import pallas as pl


def _pos_body(t_ref, o_ref):
    rids = lax.broadcasted_iota(jnp.int32, (1000, 32), 0)
    cids = lax.broadcasted_iota(jnp.int32, (1000, 32), 1)
    sel = (cids == rids % 10)
    sel |= (cids == 10 + (rids // 10) % 10)
    sel |= (cids == 20 + rids // 100)
    s = sel.astype(jnp.float32)
    pos = jax.lax.dot_general(
        s, t_ref[...],
        dimension_numbers=(((1,), (0,)), ((), ())),
        preferred_element_type=jnp.float32,
    )
    bb = o_ref.shape[0]
    o_ref[...] = jnp.broadcast_to(pos[None], (bb, 1000, 256))


def kernel(x, row_embed, col_embed, time_embed):
    bs, frame_num, h, w = x.shape[:4]
    d4 = row_embed.shape[1]
    d2 = time_embed.shape[1]
    d = 2 * d4 + d2

    t = jnp.zeros((32, d), jnp.float32)
    t = t.at[0:10, 0:d4].set(row_embed)
    t = t.at[10:20, d4:2 * d4].set(col_embed)
    t = t.at[20:30, 2 * d4:d].set(time_embed)

    bb = 8
    out = pl.pallas_call(
        _pos_body,
        grid=(bs // bb,),
        in_specs=[pl.BlockSpec((32, d), lambda i: (0, 0))],
        out_specs=pl.BlockSpec((bb, frame_num * h * w, d), lambda i: (i, 0, 0)),
        out_shape=jax.ShapeDtypeStruct((bs, frame_num * h * w, d), jnp.float32),
    )(t)
    return out.reshape(bs, frame_num, h, w, d)
